# baseline (speedup 1.0000x reference)
import functools

import jax
import jax.numpy as jnp
from jax import lax
from jax.experimental import pallas as pl
from jax.experimental.pallas import tpu as pltpu
from jax.experimental.pallas import tpu_sc as plsc

_D = 32
_NC = 2
_NS = 16
_NW = _NC * _NS
_CR = 32
_NB = 3


@functools.cache
def _make_gather(R: int, C: int):
    assert R % (_NW * _CR) == 0
    r_per_w = R // _NW
    n_chunks = r_per_w // _CR
    mesh = plsc.VectorSubcoreMesh(core_axis_name="c", subcore_axis_name="s")

    @functools.partial(
        pl.kernel,
        out_type=jax.ShapeDtypeStruct((R, C, _D), jnp.float32),
        mesh=mesh,
        compiler_params=pltpu.CompilerParams(use_tc_tiling_on_sc=False),
        scratch_types=[
            pltpu.VMEM((r_per_w, C), jnp.int32),
            pltpu.VMEM((_NB, _CR, C, _D), jnp.float32),
            [pltpu.SemaphoreType.DMA] * _NB,
            [pltpu.SemaphoreType.DMA] * _NB,
        ],
    )
    def gather_kernel(x_hbm, table_hbm, out_hbm, idx_v, rows_v, sem_g, sem_o):
        wid = lax.axis_index("s") * _NC + lax.axis_index("c")
        r0 = wid * r_per_w
        pltpu.sync_copy(x_hbm.at[pl.ds(r0, r_per_w)], idx_v)

        gathers = [None] * n_chunks
        outs = [None] * n_chunks

        def start_gathers(c, b):
            return [
                pltpu.async_copy(
                    table_hbm.at[idx_v.at[c * _CR + i]],
                    rows_v.at[b, i], sem_g[b])
                for i in range(_CR)
            ]

        def start_writeback(c, b):
            for d in gathers[c]:
                d.wait()
            return pltpu.async_copy(
                rows_v.at[b],
                out_hbm.at[pl.ds(r0 + c * _CR, _CR)],
                sem_o[b])

        for c in range(n_chunks):
            b = c % _NB
            if c >= _NB:
                outs[c - _NB].wait()
            gathers[c] = start_gathers(c, b)
            if c >= 1:
                outs[c - 1] = start_writeback(c - 1, (c - 1) % _NB)
        c = n_chunks - 1
        outs[c] = start_writeback(c, c % _NB)
        for c in range(max(0, n_chunks - _NB), n_chunks):
            outs[c].wait()

    return gather_kernel


def kernel(x, table):
    if x.dtype != jnp.int32:
        x = x.astype(jnp.int32)
    return _make_gather(x.shape[0], x.shape[1])(x, table)


# speedup vs baseline: 1.0066x; 1.0066x over previous
import functools

import jax
import jax.numpy as jnp
from jax import lax
from jax.experimental import pallas as pl
from jax.experimental.pallas import tpu as pltpu
from jax.experimental.pallas import tpu_sc as plsc

_D = 32
_NC = 2
_NS = 16
_NW = _NC * _NS
_CH = 1024
_NB = 3


@functools.cache
def _make_gather(B: int):
    assert B % (_NW * _CH) == 0
    b_per_w = B // _NW
    n_chunks = b_per_w // _CH
    mesh = plsc.VectorSubcoreMesh(core_axis_name="c", subcore_axis_name="s")

    @functools.partial(
        pl.kernel,
        out_type=jax.ShapeDtypeStruct((B, _D), jnp.float32),
        mesh=mesh,
        compiler_params=pltpu.CompilerParams(use_tc_tiling_on_sc=False),
        scratch_types=[
            pltpu.VMEM((n_chunks, _CH), jnp.int32),
            pltpu.VMEM((_NB, _CH, _D), jnp.float32),
            [pltpu.SemaphoreType.DMA] * _NB,
            [pltpu.SemaphoreType.DMA] * _NB,
        ],
    )
    def gather_kernel(idx_hbm, table_hbm, out_hbm, idx_v, rows_v, sem_g, sem_o):
        wid = lax.axis_index("s") * _NC + lax.axis_index("c")
        base_w = wid * b_per_w
        pltpu.sync_copy(idx_hbm.at[wid], idx_v)

        gathers = [None] * n_chunks
        outs = [None] * n_chunks
        for c in range(n_chunks):
            b = c % _NB
            if c >= _NB:
                outs[c - _NB].wait()
            gathers[c] = pltpu.async_copy(
                table_hbm.at[idx_v.at[c]], rows_v.at[b], sem_g[b])
            if c >= 1:
                bp = (c - 1) % _NB
                gathers[c - 1].wait()
                outs[c - 1] = pltpu.async_copy(
                    rows_v.at[bp],
                    out_hbm.at[pl.ds(base_w + (c - 1) * _CH, _CH)],
                    sem_o[bp])
        c = n_chunks - 1
        gathers[c].wait()
        outs[c] = pltpu.async_copy(
            rows_v.at[c % _NB],
            out_hbm.at[pl.ds(base_w + c * _CH, _CH)],
            sem_o[c % _NB])
        for c in range(max(0, n_chunks - _NB), n_chunks):
            outs[c].wait()

    return gather_kernel


def kernel(x, table):
    lead_shape = x.shape
    idx = x.reshape(-1).astype(jnp.int32)
    B = idx.shape[0]
    idx3 = idx.reshape(_NW, B // (_NW * _CH), _CH)
    out = _make_gather(B)(idx3, table)
    return out.reshape(*lead_shape, _D)


# speedup vs baseline: 1.0077x; 1.0011x over previous
import functools

import jax
import jax.numpy as jnp
from jax import lax
from jax.experimental import pallas as pl
from jax.experimental.pallas import tpu as pltpu
from jax.experimental.pallas import tpu_sc as plsc

_D = 32
_NC = 2
_NS = 16
_NW = _NC * _NS
_CH = 1024
_NB = 3


@functools.cache
def _make_gather(B: int):
    assert B % (_NW * _CH) == 0
    b_per_w = B // _NW
    n_chunks = b_per_w // _CH
    mesh = plsc.VectorSubcoreMesh(core_axis_name="c", subcore_axis_name="s")

    @functools.partial(
        pl.kernel,
        out_type=jax.ShapeDtypeStruct((B, _D), jnp.float32),
        mesh=mesh,
        compiler_params=pltpu.CompilerParams(use_tc_tiling_on_sc=False),
        scratch_types=[
            pltpu.VMEM((b_per_w,), jnp.int32),
            pltpu.VMEM((_NB, _CH, _D), jnp.float32),
            [pltpu.SemaphoreType.DMA] * _NB,
            [pltpu.SemaphoreType.DMA] * _NB,
        ],
    )
    def gather_kernel(idx_hbm, table_hbm, out_hbm, idx_v, rows_v, sem_g, sem_o):
        wid = lax.axis_index("s") * _NC + lax.axis_index("c")
        base_w = wid * b_per_w
        pltpu.sync_copy(idx_hbm.at[pl.ds(base_w, b_per_w)], idx_v)

        gathers = [None] * n_chunks
        outs = [None] * n_chunks
        for c in range(n_chunks):
            b = c % _NB
            if c >= _NB:
                outs[c - _NB].wait()
            gathers[c] = pltpu.async_copy(
                table_hbm.at[idx_v.at[pl.ds(c * _CH, _CH)]], rows_v.at[b], sem_g[b])
            if c >= 1:
                bp = (c - 1) % _NB
                gathers[c - 1].wait()
                outs[c - 1] = pltpu.async_copy(
                    rows_v.at[bp],
                    out_hbm.at[pl.ds(base_w + (c - 1) * _CH, _CH)],
                    sem_o[bp])
        c = n_chunks - 1
        gathers[c].wait()
        outs[c] = pltpu.async_copy(
            rows_v.at[c % _NB],
            out_hbm.at[pl.ds(base_w + c * _CH, _CH)],
            sem_o[c % _NB])
        for c in range(max(0, n_chunks - _NB), n_chunks):
            outs[c].wait()

    return gather_kernel


def kernel(x, table):
    lead_shape = x.shape
    idx = x.reshape(-1).astype(jnp.int32)
    B = idx.shape[0]
    out = _make_gather(B)(idx, table)
    return out.reshape(*lead_shape, _D)


# speedup vs baseline: 1.0080x; 1.0003x over previous
import functools

import jax
import jax.numpy as jnp
from jax import lax
from jax.experimental import pallas as pl
from jax.experimental.pallas import tpu as pltpu
from jax.experimental.pallas import tpu_sc as plsc

_D = 32
_NC = 2
_NS = 16
_NW = _NC * _NS
_CH = 1664
_NB = 2


@functools.cache
def _make_gather(B: int):
    assert B % (_NW * _CH) == 0
    b_per_w = B // _NW
    n_chunks = b_per_w // _CH
    mesh = plsc.VectorSubcoreMesh(core_axis_name="c", subcore_axis_name="s")

    @functools.partial(
        pl.kernel,
        out_type=jax.ShapeDtypeStruct((B, _D), jnp.float32),
        mesh=mesh,
        compiler_params=pltpu.CompilerParams(use_tc_tiling_on_sc=False),
        scratch_types=[
            pltpu.VMEM((b_per_w,), jnp.int32),
            pltpu.VMEM((_NB, _CH, _D), jnp.float32),
            [pltpu.SemaphoreType.DMA] * _NB,
            [pltpu.SemaphoreType.DMA] * _NB,
        ],
    )
    def gather_kernel(idx_hbm, table_hbm, out_hbm, idx_v, rows_v, sem_g, sem_o):
        wid = lax.axis_index("s") * _NC + lax.axis_index("c")
        base_w = wid * b_per_w
        pltpu.sync_copy(idx_hbm.at[pl.ds(base_w, b_per_w)], idx_v)

        gathers = [None] * n_chunks
        outs = [None] * n_chunks
        for c in range(n_chunks):
            b = c % _NB
            if c >= _NB:
                outs[c - _NB].wait()
            gathers[c] = pltpu.async_copy(
                table_hbm.at[idx_v.at[pl.ds(c * _CH, _CH)]], rows_v.at[b], sem_g[b])
            if c >= 1:
                bp = (c - 1) % _NB
                gathers[c - 1].wait()
                outs[c - 1] = pltpu.async_copy(
                    rows_v.at[bp],
                    out_hbm.at[pl.ds(base_w + (c - 1) * _CH, _CH)],
                    sem_o[bp])
        c = n_chunks - 1
        gathers[c].wait()
        outs[c] = pltpu.async_copy(
            rows_v.at[c % _NB],
            out_hbm.at[pl.ds(base_w + c * _CH, _CH)],
            sem_o[c % _NB])
        for c in range(max(0, n_chunks - _NB), n_chunks):
            outs[c].wait()

    return gather_kernel


def kernel(x, table):
    lead_shape = x.shape
    idx = x.reshape(-1).astype(jnp.int32)
    B = idx.shape[0]
    out = _make_gather(B)(idx, table)
    return out.reshape(*lead_shape, _D)
